# baseline (speedup 1.0000x reference)
import functools

import jax
import jax.numpy as jnp
from jax import lax
from jax.experimental import pallas as pl
from jax.experimental.pallas import tpu as pltpu
from jax.experimental.pallas import tpu_sc as plsc

N_ROWS = 320000
D = 128
NUM_GRAPHS = 512
LANES = 16
N_SUBCORES = 16
NC = 2
ACC = NUM_GRAPHS * LANES
G_PER_TILE = NUM_GRAPHS // N_SUBCORES

TC_SHARE = 225280
SC_SHARE = N_ROWS - TC_SHARE
NW = NC * N_SUBCORES
ROWS_TILE = SC_SHARE // NW
CROWS = 296
NCHUNK = ROWS_TILE // CROWS

_R = 16
_TC_VIEW = TC_SHARE // D


def _energy_body(x_ref, o_ref):
    x = x_ref[...]
    o_ref[...] = 0.5 * jnp.sum(x * x, axis=-1)


_energy_tc = pl.pallas_call(
    _energy_body,
    grid=(_TC_VIEW // _R,),
    in_specs=[pl.BlockSpec((_R, D, D), lambda i: (i, 0, 0))],
    out_specs=pl.BlockSpec((_R, D), lambda i: (i, 0)),
    out_shape=jax.ShapeDtypeStruct((_TC_VIEW, D), jnp.float32),
    compiler_params=pltpu.CompilerParams(
        dimension_semantics=("arbitrary",),
    ),
)

_mesh1 = plsc.VectorSubcoreMesh(
    core_axis_name="c", subcore_axis_name="s", num_cores=1
)
_mesh2 = plsc.VectorSubcoreMesh(
    core_axis_name="c", subcore_axis_name="s", num_cores=2
)


@functools.partial(
    pl.kernel,
    mesh=_mesh2,
    out_type=jax.ShapeDtypeStruct((NC, NUM_GRAPHS), jnp.float32),
    scratch_types=[
        pltpu.VMEM((CROWS * D,), jnp.float32),
        pltpu.VMEM((CROWS * D,), jnp.float32),
        pltpu.VMEM((ROWS_TILE + LANES,), jnp.int32),
        pltpu.VMEM((ACC,), jnp.float32),
        pltpu.VMEM((G_PER_TILE * LANES,), jnp.float32),
        pltpu.VMEM((N_SUBCORES, G_PER_TILE * LANES), jnp.float32),
        pltpu.VMEM((G_PER_TILE,), jnp.float32),
        pltpu.VMEM_SHARED((N_SUBCORES, ACC), jnp.float32),
        pltpu.SemaphoreType.DMA,
        pltpu.SemaphoreType.DMA,
    ],
    compiler_params=pltpu.CompilerParams(needs_layout_passes=False),
)
def _energy_scatter_sc(x_hbm, b_hbm, out_hbm, xb0, xb1, b_v, acc_v, sum_v,
                       stage_v, res_v, shared, sem0, sem1):
    cid = lax.axis_index("c")
    sid = lax.axis_index("s")
    wid = sid * NC + cid
    rbase = TC_SHARE + wid * ROWS_TILE

    pltpu.sync_copy(b_hbm.at[pl.ds(rbase, ROWS_TILE)],
                    b_v.at[pl.ds(0, ROWS_TILE)])

    zeros16 = jnp.zeros((LANES,), jnp.float32)

    @plsc.parallel_loop(0, ACC // LANES, unroll=8)
    def _zero(i):
        acc_v[pl.ds(i * LANES, LANES)] = zeros16

    lane = lax.iota(jnp.int32, LANES)
    bufs = (xb0, xb1)
    sems = (sem0, sem1)
    cps = [None, None]
    cps[0] = pltpu.async_copy(
        x_hbm.at[pl.ds(rbase * D, CROWS * D)], xb0, sem0
    )
    for ch in range(NCHUNK):
        cur = ch % 2
        if ch + 1 < NCHUNK:
            nxt = 1 - cur
            cps[nxt] = pltpu.async_copy(
                x_hbm.at[pl.ds((rbase + (ch + 1) * CROWS) * D, CROWS * D)],
                bufs[nxt], sems[nxt],
            )
        cps[cur].wait()
        xb = bufs[cur]
        boff = ch * CROWS

        @plsc.parallel_loop(0, CROWS, unroll=4)
        def _rows(r):
            bval = b_v[pl.ds(boff + r, LANES)][0]
            off = r * D
            v = xb[pl.ds(off, LANES)]
            sq = v * v
            for k in range(1, D // LANES):
                v = xb[pl.ds(off + k * LANES, LANES)]
                sq = sq + v * v
            plsc.addupdate_scatter(acc_v, [bval * LANES + lane], sq * 0.5)

    pltpu.sync_copy(acc_v, shared.at[sid])
    plsc.subcore_barrier()

    goff = sid * G_PER_TILE * LANES
    pltpu.sync_copy(shared.at[:, pl.ds(goff, G_PER_TILE * LANES)], stage_v)

    @plsc.parallel_loop(0, G_PER_TILE, unroll=4)
    def _fold_tiles(c):
        s = pl.ds(c * LANES, LANES)
        tot = stage_v[0, s]
        for t in range(1, N_SUBCORES):
            tot = tot + stage_v[t, s]
        sum_v[s] = tot

    for c in range(G_PER_TILE // LANES):
        addr = c * LANES * LANES + lane * LANES
        tot = plsc.load_gather(sum_v, [addr])
        for l in range(1, LANES):
            tot = tot + plsc.load_gather(sum_v, [addr + l])
        res_v[pl.ds(c * LANES, LANES)] = tot

    pltpu.sync_copy(
        res_v, out_hbm.at[cid, pl.ds(sid * G_PER_TILE, G_PER_TILE)]
    )


_CHUNK_B = TC_SHARE // N_SUBCORES


@functools.partial(
    pl.kernel,
    mesh=_mesh1,
    out_type=jax.ShapeDtypeStruct((NUM_GRAPHS,), jnp.float32),
    scratch_types=[
        pltpu.VMEM((_CHUNK_B,), jnp.float32),
        pltpu.VMEM((_CHUNK_B,), jnp.int32),
        pltpu.VMEM((ACC,), jnp.float32),
        pltpu.VMEM((G_PER_TILE * LANES,), jnp.float32),
        pltpu.VMEM((N_SUBCORES, G_PER_TILE * LANES), jnp.float32),
        pltpu.VMEM((G_PER_TILE,), jnp.float32),
        pltpu.VMEM((G_PER_TILE,), jnp.float32),
        pltpu.VMEM((G_PER_TILE,), jnp.float32),
        pltpu.VMEM_SHARED((N_SUBCORES, ACC), jnp.float32),
    ],
    compiler_params=pltpu.CompilerParams(needs_layout_passes=False),
)
def _scatter_tc(e_hbm, b_hbm, prev_hbm, out_hbm, e_v, b_v, acc_v, sum_v,
                stage_v, res_v, prev0_v, prev1_v, shared):
    sid = lax.axis_index("s")
    base = sid * _CHUNK_B

    pltpu.sync_copy(e_hbm.at[pl.ds(base, _CHUNK_B)], e_v)
    pltpu.sync_copy(b_hbm.at[pl.ds(base, _CHUNK_B)], b_v)
    pltpu.sync_copy(prev_hbm.at[0, pl.ds(sid * G_PER_TILE, G_PER_TILE)],
                    prev0_v)
    pltpu.sync_copy(prev_hbm.at[1, pl.ds(sid * G_PER_TILE, G_PER_TILE)],
                    prev1_v)

    zeros16 = jnp.zeros((LANES,), jnp.float32)

    @plsc.parallel_loop(0, ACC // LANES, unroll=8)
    def _zero(i):
        acc_v[pl.ds(i * LANES, LANES)] = zeros16

    lane = lax.iota(jnp.int32, LANES)

    @plsc.parallel_loop(0, _CHUNK_B // LANES, unroll=8)
    def _accum(i):
        s = pl.ds(i * LANES, LANES)
        idx = b_v[s]
        ev = e_v[s]
        plsc.addupdate_scatter(acc_v, [idx * LANES + lane], ev)

    pltpu.sync_copy(acc_v, shared.at[sid])
    plsc.subcore_barrier()

    goff = sid * G_PER_TILE * LANES
    pltpu.sync_copy(shared.at[:, pl.ds(goff, G_PER_TILE * LANES)], stage_v)

    @plsc.parallel_loop(0, G_PER_TILE, unroll=4)
    def _fold_tiles(c):
        s = pl.ds(c * LANES, LANES)
        tot = stage_v[0, s]
        for t in range(1, N_SUBCORES):
            tot = tot + stage_v[t, s]
        sum_v[s] = tot

    for c in range(G_PER_TILE // LANES):
        addr = c * LANES * LANES + lane * LANES
        tot = plsc.load_gather(sum_v, [addr])
        for l in range(1, LANES):
            tot = tot + plsc.load_gather(sum_v, [addr + l])
        s = pl.ds(c * LANES, LANES)
        res_v[s] = tot + prev0_v[s] + prev1_v[s]

    pltpu.sync_copy(res_v, out_hbm.at[pl.ds(sid * G_PER_TILE, G_PER_TILE)])


def kernel(X, batch, num_graphs):
    del num_graphs
    b = batch.astype(jnp.int32)
    part_sc = _energy_scatter_sc(X.reshape(-1), b)
    e = _energy_tc(X.reshape(2500, D, D)).reshape(-1)
    return _scatter_tc(e, b, part_sc)


# speedup vs baseline: 1.4404x; 1.4404x over previous
import functools

import jax
import jax.numpy as jnp
from jax import lax
from jax.experimental import pallas as pl
from jax.experimental.pallas import tpu as pltpu
from jax.experimental.pallas import tpu_sc as plsc

N_ROWS = 320000
D = 128
NUM_GRAPHS = 512
LANES = 16
N_SUBCORES = 16
NC = 2
ACC = NUM_GRAPHS * LANES
G_PER_TILE = NUM_GRAPHS // N_SUBCORES

TC_SHARE = 229376
SC_SHARE = N_ROWS - TC_SHARE
NW = NC * N_SUBCORES
ROWS_TILE = SC_SHARE // NW
CROWS = 236
NCHUNK = ROWS_TILE // CROWS

_R = 64
_TC_VIEW = TC_SHARE // D


def _energy_body(x_ref, o_ref):
    x = x_ref[...]
    o_ref[...] = 0.5 * jnp.sum(x * x, axis=-1)


_energy_tc = pl.pallas_call(
    _energy_body,
    grid=(_TC_VIEW // _R,),
    in_specs=[pl.BlockSpec((_R, D, D), lambda i: (i, 0, 0))],
    out_specs=pl.BlockSpec((_R, D), lambda i: (i, 0)),
    out_shape=jax.ShapeDtypeStruct((_TC_VIEW, D), jnp.float32),
    compiler_params=pltpu.CompilerParams(
        dimension_semantics=("arbitrary",),
    ),
)

_mesh1 = plsc.VectorSubcoreMesh(
    core_axis_name="c", subcore_axis_name="s", num_cores=1
)
_mesh2 = plsc.VectorSubcoreMesh(
    core_axis_name="c", subcore_axis_name="s", num_cores=2
)


@functools.partial(
    pl.kernel,
    mesh=_mesh2,
    out_type=jax.ShapeDtypeStruct((NC, NUM_GRAPHS), jnp.float32),
    scratch_types=[
        pltpu.VMEM((CROWS * D,), jnp.float32),
        pltpu.VMEM((CROWS * D,), jnp.float32),
        pltpu.VMEM((ROWS_TILE + LANES,), jnp.int32),
        pltpu.VMEM((ACC,), jnp.float32),
        pltpu.VMEM((G_PER_TILE * LANES,), jnp.float32),
        pltpu.VMEM((N_SUBCORES, G_PER_TILE * LANES), jnp.float32),
        pltpu.VMEM((G_PER_TILE,), jnp.float32),
        pltpu.VMEM_SHARED((N_SUBCORES, ACC), jnp.float32),
        pltpu.SemaphoreType.DMA,
        pltpu.SemaphoreType.DMA,
    ],
    compiler_params=pltpu.CompilerParams(needs_layout_passes=False),
)
def _energy_scatter_sc(x_hbm, b_hbm, out_hbm, xb0, xb1, b_v, acc_v, sum_v,
                       stage_v, res_v, shared, sem0, sem1):
    cid = lax.axis_index("c")
    sid = lax.axis_index("s")
    wid = sid * NC + cid
    rbase = TC_SHARE + wid * ROWS_TILE

    pltpu.sync_copy(b_hbm.at[pl.ds(rbase, ROWS_TILE)],
                    b_v.at[pl.ds(0, ROWS_TILE)])

    zeros16 = jnp.zeros((LANES,), jnp.float32)

    @plsc.parallel_loop(0, ACC // LANES, unroll=8)
    def _zero(i):
        acc_v[pl.ds(i * LANES, LANES)] = zeros16

    lane = lax.iota(jnp.int32, LANES)
    bufs = (xb0, xb1)
    sems = (sem0, sem1)
    cps = [None, None]
    cps[0] = pltpu.async_copy(
        x_hbm.at[pl.ds(rbase * D, CROWS * D)], xb0, sem0
    )
    for ch in range(NCHUNK):
        cur = ch % 2
        if ch + 1 < NCHUNK:
            nxt = 1 - cur
            cps[nxt] = pltpu.async_copy(
                x_hbm.at[pl.ds((rbase + (ch + 1) * CROWS) * D, CROWS * D)],
                bufs[nxt], sems[nxt],
            )
        cps[cur].wait()
        xb = bufs[cur]
        boff = ch * CROWS

        @plsc.parallel_loop(0, CROWS, unroll=4)
        def _rows(r):
            bval = b_v[pl.ds(boff + r, LANES)][0]
            off = r * D
            sq = [None] * (D // LANES)
            for k in range(D // LANES):
                v = xb[pl.ds(off + k * LANES, LANES)]
                sq[k] = v * v
            n = D // LANES
            while n > 1:
                for k in range(n // 2):
                    sq[k] = sq[2 * k] + sq[2 * k + 1]
                n //= 2
            plsc.addupdate_scatter(acc_v, [bval * LANES + lane],
                                   sq[0] * 0.5)

    pltpu.sync_copy(acc_v, shared.at[sid])
    plsc.subcore_barrier()

    goff = sid * G_PER_TILE * LANES
    pltpu.sync_copy(shared.at[:, pl.ds(goff, G_PER_TILE * LANES)], stage_v)

    @plsc.parallel_loop(0, G_PER_TILE, unroll=4)
    def _fold_tiles(c):
        s = pl.ds(c * LANES, LANES)
        tot = stage_v[0, s]
        for t in range(1, N_SUBCORES):
            tot = tot + stage_v[t, s]
        sum_v[s] = tot

    for c in range(G_PER_TILE // LANES):
        addr = c * LANES * LANES + lane * LANES
        tot = plsc.load_gather(sum_v, [addr])
        for l in range(1, LANES):
            tot = tot + plsc.load_gather(sum_v, [addr + l])
        res_v[pl.ds(c * LANES, LANES)] = tot

    pltpu.sync_copy(
        res_v, out_hbm.at[cid, pl.ds(sid * G_PER_TILE, G_PER_TILE)]
    )


_CHUNK_B = TC_SHARE // N_SUBCORES


@functools.partial(
    pl.kernel,
    mesh=_mesh1,
    out_type=jax.ShapeDtypeStruct((NUM_GRAPHS,), jnp.float32),
    scratch_types=[
        pltpu.VMEM((_CHUNK_B,), jnp.float32),
        pltpu.VMEM((_CHUNK_B,), jnp.int32),
        pltpu.VMEM((ACC,), jnp.float32),
        pltpu.VMEM((G_PER_TILE * LANES,), jnp.float32),
        pltpu.VMEM((N_SUBCORES, G_PER_TILE * LANES), jnp.float32),
        pltpu.VMEM((G_PER_TILE,), jnp.float32),
        pltpu.VMEM((G_PER_TILE,), jnp.float32),
        pltpu.VMEM((G_PER_TILE,), jnp.float32),
        pltpu.VMEM_SHARED((N_SUBCORES, ACC), jnp.float32),
    ],
    compiler_params=pltpu.CompilerParams(needs_layout_passes=False),
)
def _scatter_tc(e_hbm, b_hbm, prev_hbm, out_hbm, e_v, b_v, acc_v, sum_v,
                stage_v, res_v, prev0_v, prev1_v, shared):
    sid = lax.axis_index("s")
    base = sid * _CHUNK_B

    pltpu.sync_copy(e_hbm.at[pl.ds(base, _CHUNK_B)], e_v)
    pltpu.sync_copy(b_hbm.at[pl.ds(base, _CHUNK_B)], b_v)
    pltpu.sync_copy(prev_hbm.at[0, pl.ds(sid * G_PER_TILE, G_PER_TILE)],
                    prev0_v)
    pltpu.sync_copy(prev_hbm.at[1, pl.ds(sid * G_PER_TILE, G_PER_TILE)],
                    prev1_v)

    zeros16 = jnp.zeros((LANES,), jnp.float32)

    @plsc.parallel_loop(0, ACC // LANES, unroll=8)
    def _zero(i):
        acc_v[pl.ds(i * LANES, LANES)] = zeros16

    lane = lax.iota(jnp.int32, LANES)

    @plsc.parallel_loop(0, _CHUNK_B // LANES, unroll=8)
    def _accum(i):
        s = pl.ds(i * LANES, LANES)
        idx = b_v[s]
        ev = e_v[s]
        plsc.addupdate_scatter(acc_v, [idx * LANES + lane], ev)

    pltpu.sync_copy(acc_v, shared.at[sid])
    plsc.subcore_barrier()

    goff = sid * G_PER_TILE * LANES
    pltpu.sync_copy(shared.at[:, pl.ds(goff, G_PER_TILE * LANES)], stage_v)

    @plsc.parallel_loop(0, G_PER_TILE, unroll=4)
    def _fold_tiles(c):
        s = pl.ds(c * LANES, LANES)
        tot = stage_v[0, s]
        for t in range(1, N_SUBCORES):
            tot = tot + stage_v[t, s]
        sum_v[s] = tot

    for c in range(G_PER_TILE // LANES):
        addr = c * LANES * LANES + lane * LANES
        tot = plsc.load_gather(sum_v, [addr])
        for l in range(1, LANES):
            tot = tot + plsc.load_gather(sum_v, [addr + l])
        s = pl.ds(c * LANES, LANES)
        res_v[s] = tot + prev0_v[s] + prev1_v[s]

    pltpu.sync_copy(res_v, out_hbm.at[pl.ds(sid * G_PER_TILE, G_PER_TILE)])


def kernel(X, batch, num_graphs):
    del num_graphs
    b = batch.astype(jnp.int32)
    part_sc = _energy_scatter_sc(X.reshape(-1), b)
    e = _energy_tc(X.reshape(2500, D, D)).reshape(-1)
    return _scatter_tc(e, b, part_sc)


# speedup vs baseline: 1.4630x; 1.0157x over previous
import functools

import jax
import jax.numpy as jnp
from jax import lax
from jax.experimental import pallas as pl
from jax.experimental.pallas import tpu as pltpu
from jax.experimental.pallas import tpu_sc as plsc

N_ROWS = 320000
D = 128
NUM_GRAPHS = 512
LANES = 16
N_SUBCORES = 16
NC = 2
ACC = NUM_GRAPHS * LANES
G_PER_TILE = NUM_GRAPHS // N_SUBCORES

TC_SHARE = 212992
SC_SHARE = N_ROWS - TC_SHARE
NW = NC * N_SUBCORES
ROWS_TILE = SC_SHARE // NW
CROWS = 209
NCHUNK = ROWS_TILE // CROWS

_R = 64
_TC_VIEW = TC_SHARE // D


def _energy_body(x_ref, o_ref):
    x = x_ref[...]
    o_ref[...] = 0.5 * jnp.sum(x * x, axis=-1)


_energy_tc = pl.pallas_call(
    _energy_body,
    grid=(_TC_VIEW // _R,),
    in_specs=[pl.BlockSpec((_R, D, D), lambda i: (i, 0, 0))],
    out_specs=pl.BlockSpec((_R, D), lambda i: (i, 0)),
    out_shape=jax.ShapeDtypeStruct((_TC_VIEW, D), jnp.float32),
    compiler_params=pltpu.CompilerParams(
        dimension_semantics=("arbitrary",),
    ),
)

_mesh1 = plsc.VectorSubcoreMesh(
    core_axis_name="c", subcore_axis_name="s", num_cores=1
)
_mesh2 = plsc.VectorSubcoreMesh(
    core_axis_name="c", subcore_axis_name="s", num_cores=2
)


@functools.partial(
    pl.kernel,
    mesh=_mesh2,
    out_type=jax.ShapeDtypeStruct((NC, NUM_GRAPHS), jnp.float32),
    scratch_types=[
        pltpu.VMEM((CROWS * D,), jnp.float32),
        pltpu.VMEM((CROWS * D,), jnp.float32),
        pltpu.VMEM((ROWS_TILE + LANES,), jnp.int32),
        pltpu.VMEM((ACC,), jnp.float32),
        pltpu.VMEM((G_PER_TILE * LANES,), jnp.float32),
        pltpu.VMEM((N_SUBCORES, G_PER_TILE * LANES), jnp.float32),
        pltpu.VMEM((G_PER_TILE,), jnp.float32),
        pltpu.VMEM_SHARED((N_SUBCORES, ACC), jnp.float32),
        pltpu.SemaphoreType.DMA,
        pltpu.SemaphoreType.DMA,
    ],
    compiler_params=pltpu.CompilerParams(needs_layout_passes=False),
)
def _energy_scatter_sc(x_hbm, b_hbm, out_hbm, xb0, xb1, b_v, acc_v, sum_v,
                       stage_v, res_v, shared, sem0, sem1):
    cid = lax.axis_index("c")
    sid = lax.axis_index("s")
    wid = sid * NC + cid
    rbase = TC_SHARE + wid * ROWS_TILE

    pltpu.sync_copy(b_hbm.at[pl.ds(rbase, ROWS_TILE)],
                    b_v.at[pl.ds(0, ROWS_TILE)])

    zeros16 = jnp.zeros((LANES,), jnp.float32)

    @plsc.parallel_loop(0, ACC // LANES, unroll=8)
    def _zero(i):
        acc_v[pl.ds(i * LANES, LANES)] = zeros16

    lane = lax.iota(jnp.int32, LANES)
    bufs = (xb0, xb1)
    sems = (sem0, sem1)
    cps = [None, None]
    cps[0] = pltpu.async_copy(
        x_hbm.at[pl.ds(rbase * D, CROWS * D)], xb0, sem0
    )
    for ch in range(NCHUNK):
        cur = ch % 2
        if ch + 1 < NCHUNK:
            nxt = 1 - cur
            cps[nxt] = pltpu.async_copy(
                x_hbm.at[pl.ds((rbase + (ch + 1) * CROWS) * D, CROWS * D)],
                bufs[nxt], sems[nxt],
            )
        cps[cur].wait()
        xb = bufs[cur]
        boff = ch * CROWS

        @plsc.parallel_loop(0, CROWS, unroll=11)
        def _rows(r):
            bval = b_v[pl.ds(boff + r, LANES)][0]
            off = r * D
            sq = [None] * (D // LANES)
            for k in range(D // LANES):
                v = xb[pl.ds(off + k * LANES, LANES)]
                sq[k] = v * v
            n = D // LANES
            while n > 1:
                for k in range(n // 2):
                    sq[k] = sq[2 * k] + sq[2 * k + 1]
                n //= 2
            plsc.addupdate_scatter(acc_v, [bval * LANES + lane],
                                   sq[0] * 0.5)

    pltpu.sync_copy(acc_v, shared.at[sid])
    plsc.subcore_barrier()

    goff = sid * G_PER_TILE * LANES
    pltpu.sync_copy(shared.at[:, pl.ds(goff, G_PER_TILE * LANES)], stage_v)

    @plsc.parallel_loop(0, G_PER_TILE, unroll=4)
    def _fold_tiles(c):
        s = pl.ds(c * LANES, LANES)
        tot = stage_v[0, s]
        for t in range(1, N_SUBCORES):
            tot = tot + stage_v[t, s]
        sum_v[s] = tot

    for c in range(G_PER_TILE // LANES):
        addr = c * LANES * LANES + lane * LANES
        tot = plsc.load_gather(sum_v, [addr])
        for l in range(1, LANES):
            tot = tot + plsc.load_gather(sum_v, [addr + l])
        res_v[pl.ds(c * LANES, LANES)] = tot

    pltpu.sync_copy(
        res_v, out_hbm.at[cid, pl.ds(sid * G_PER_TILE, G_PER_TILE)]
    )


_CHUNK_B = TC_SHARE // N_SUBCORES


@functools.partial(
    pl.kernel,
    mesh=_mesh1,
    out_type=jax.ShapeDtypeStruct((NUM_GRAPHS,), jnp.float32),
    scratch_types=[
        pltpu.VMEM((_CHUNK_B,), jnp.float32),
        pltpu.VMEM((_CHUNK_B,), jnp.int32),
        pltpu.VMEM((ACC,), jnp.float32),
        pltpu.VMEM((G_PER_TILE * LANES,), jnp.float32),
        pltpu.VMEM((N_SUBCORES, G_PER_TILE * LANES), jnp.float32),
        pltpu.VMEM((G_PER_TILE,), jnp.float32),
        pltpu.VMEM((G_PER_TILE,), jnp.float32),
        pltpu.VMEM((G_PER_TILE,), jnp.float32),
        pltpu.VMEM_SHARED((N_SUBCORES, ACC), jnp.float32),
    ],
    compiler_params=pltpu.CompilerParams(needs_layout_passes=False),
)
def _scatter_tc(e_hbm, b_hbm, prev_hbm, out_hbm, e_v, b_v, acc_v, sum_v,
                stage_v, res_v, prev0_v, prev1_v, shared):
    sid = lax.axis_index("s")
    base = sid * _CHUNK_B

    pltpu.sync_copy(e_hbm.at[pl.ds(base, _CHUNK_B)], e_v)
    pltpu.sync_copy(b_hbm.at[pl.ds(base, _CHUNK_B)], b_v)
    pltpu.sync_copy(prev_hbm.at[0, pl.ds(sid * G_PER_TILE, G_PER_TILE)],
                    prev0_v)
    pltpu.sync_copy(prev_hbm.at[1, pl.ds(sid * G_PER_TILE, G_PER_TILE)],
                    prev1_v)

    zeros16 = jnp.zeros((LANES,), jnp.float32)

    @plsc.parallel_loop(0, ACC // LANES, unroll=8)
    def _zero(i):
        acc_v[pl.ds(i * LANES, LANES)] = zeros16

    lane = lax.iota(jnp.int32, LANES)

    @plsc.parallel_loop(0, _CHUNK_B // LANES, unroll=8)
    def _accum(i):
        s = pl.ds(i * LANES, LANES)
        idx = b_v[s]
        ev = e_v[s]
        plsc.addupdate_scatter(acc_v, [idx * LANES + lane], ev)

    pltpu.sync_copy(acc_v, shared.at[sid])
    plsc.subcore_barrier()

    goff = sid * G_PER_TILE * LANES
    pltpu.sync_copy(shared.at[:, pl.ds(goff, G_PER_TILE * LANES)], stage_v)

    @plsc.parallel_loop(0, G_PER_TILE, unroll=4)
    def _fold_tiles(c):
        s = pl.ds(c * LANES, LANES)
        tot = stage_v[0, s]
        for t in range(1, N_SUBCORES):
            tot = tot + stage_v[t, s]
        sum_v[s] = tot

    for c in range(G_PER_TILE // LANES):
        addr = c * LANES * LANES + lane * LANES
        tot = plsc.load_gather(sum_v, [addr])
        for l in range(1, LANES):
            tot = tot + plsc.load_gather(sum_v, [addr + l])
        s = pl.ds(c * LANES, LANES)
        res_v[s] = tot + prev0_v[s] + prev1_v[s]

    pltpu.sync_copy(res_v, out_hbm.at[pl.ds(sid * G_PER_TILE, G_PER_TILE)])


def kernel(X, batch, num_graphs):
    del num_graphs
    b = batch.astype(jnp.int32)
    part_sc = _energy_scatter_sc(X.reshape(-1), b)
    e = _energy_tc(X.reshape(2500, D, D)).reshape(-1)
    return _scatter_tc(e, b, part_sc)
